# baseline (speedup 1.0000x reference)
import jax
import jax.numpy as jnp
import numpy as np
from jax.experimental import pallas as pl
from jax.experimental.pallas import tpu as pltpu

_D = 512
_TOK_BLK = 4096
_NTOK = 4 * 8192
_GRID = _NTOK // _TOK_BLK

_W = np.zeros((64, 128), np.float32)
_W[0:3, 0] = [1.0, 2.0, 4.0]
for j, base in enumerate((16, 32, 48)):
    _W[base:base + 16, 1 + j] = [float(1 << k) for k in range(16)]
_W = _W.astype(jnp.bfloat16)


def _compute_left(x128, w):
    t = x128.shape[0]
    bits = (x128[:, 0:64] > 0.5).astype(jnp.bfloat16)
    m = jnp.dot(bits, w, preferred_element_type=jnp.float32)
    mi = m.astype(jnp.int32)

    flags = mi[:, 0:1]
    mark = jnp.bitwise_and(flags, 1) > 0
    shl = jnp.bitwise_and(flags, 2) > 0
    shr = jnp.logical_and(jnp.logical_not(shl), jnp.bitwise_and(flags, 4) > 0)
    active = jnp.logical_and(mark, jnp.logical_or(shl, shr))

    def first_set(col):
        v = mi[:, col:col + 1]
        low = jnp.bitwise_and(v, -v)
        f = low.astype(jnp.float32)
        e = jnp.right_shift(jax.lax.bitcast_convert_type(f, jnp.int32), 23) - 127
        return jnp.where(v == 0, 0, e)

    lo = first_set(1)
    hi = first_set(2)
    sa = first_set(3)

    value = lo + 16 * hi
    shl_res = jnp.bitwise_and(jnp.left_shift(value, sa), 255)
    shr_res = jnp.right_shift(value, sa)
    res = jnp.where(shl, shl_res, shr_res)
    res_lo = jnp.bitwise_and(res, 15)
    res_hi = 16 + jnp.right_shift(res, 4)

    col = jax.lax.broadcasted_iota(jnp.int32, (t, 32), 1)
    hit = jnp.logical_or(col == res_lo, col == res_hi)
    add = jnp.where(jnp.logical_and(active, hit), 2.0, 0.0).astype(x128.dtype)
    return jnp.concatenate([x128[:, 0:96], x128[:, 96:128] + add], axis=1)


def _body(x_hbm, w_ref, o_hbm, in_buf, out_buf, sem_in, sem_out, sem_cp):
    i = pl.program_id(0)
    slot = jax.lax.rem(i, 2)

    def rows(j):
        return pl.ds(j * _TOK_BLK, _TOK_BLK)

    def in_dma(j, s):
        return pltpu.make_async_copy(
            x_hbm.at[rows(j), pl.ds(0, 128)], in_buf.at[s], sem_in.at[s])

    def out_dma(j, s):
        return pltpu.make_async_copy(
            out_buf.at[s], o_hbm.at[rows(j), pl.ds(0, 128)], sem_out.at[s])

    def cp_dma(j, s):
        return pltpu.make_async_copy(
            x_hbm.at[rows(j), pl.ds(128, 384)],
            o_hbm.at[rows(j), pl.ds(128, 384)], sem_cp.at[s])

    @pl.when(i == 0)
    def _prologue():
        in_dma(0, 0).start()
        cp_dma(0, 0).start()

    @pl.when(i + 1 < _GRID)
    def _prefetch():
        in_dma(i + 1, 1 - slot).start()
        cp_dma(i + 1, 1 - slot).start()

    @pl.when(i >= 2)
    def _drain_prev_out():
        out_dma(i - 2, slot).wait()

    in_dma(i, slot).wait()
    out_buf[slot] = _compute_left(in_buf[slot], w_ref[...])
    out_dma(i, slot).start()

    @pl.when(i >= 1)
    def _drain_prev_cp():
        cp_dma(i - 1, 1 - slot).wait()

    @pl.when(i == _GRID - 1)
    def _epilogue():
        @pl.when(_GRID >= 2)
        def _():
            out_dma(i - 1, 1 - slot).wait()
        out_dma(i, slot).wait()
        cp_dma(i, slot).wait()


def kernel(x_bd):
    b, s, d = x_bd.shape
    n = b * s
    x2 = x_bd.reshape(n, d)
    out = pl.pallas_call(
        _body,
        grid=(_GRID,),
        in_specs=[
            pl.BlockSpec(memory_space=pl.ANY),
            pl.BlockSpec((64, 128), lambda i: (0, 0)),
        ],
        out_specs=pl.BlockSpec(memory_space=pl.ANY),
        out_shape=jax.ShapeDtypeStruct((n, d), x_bd.dtype),
        scratch_shapes=[
            pltpu.VMEM((2, _TOK_BLK, 128), jnp.float32),
            pltpu.VMEM((2, _TOK_BLK, 128), jnp.float32),
            pltpu.SemaphoreType.DMA((2,)),
            pltpu.SemaphoreType.DMA((2,)),
            pltpu.SemaphoreType.DMA((2,)),
        ],
        compiler_params=pltpu.CompilerParams(
            dimension_semantics=("arbitrary",)),
    )(x2, _W)
    return out.reshape(b, s, d)


# speedup vs baseline: 35.2120x; 35.2120x over previous
import jax
import jax.numpy as jnp
import numpy as np
from jax.experimental import pallas as pl
from jax.experimental.pallas import tpu as pltpu

_D = 512
_TOK_BLK = 4096

_WT = np.zeros((8, 64), np.float32)
_WT[0, 0:3] = [1.0, 2.0, 4.0]
for j, base in enumerate((16, 32, 48)):
    _WT[1 + j, base:base + 16] = [float(1 << k) for k in range(16)]
_WT = _WT.astype(jnp.bfloat16)

_I2 = (2.0 * np.eye(32, dtype=np.float32)).astype(jnp.bfloat16)


def _body(x_ref, wt_ref, i2_ref, o_ref):
    x = x_ref[...]
    t = x.shape[0]

    bits = (x[:, 0:64] > 0.5).astype(jnp.bfloat16)
    m = jax.lax.dot_general(
        wt_ref[...], bits, (((1,), (1,)), ((), ())),
        preferred_element_type=jnp.float32)
    mi = m.astype(jnp.int32)

    flags = mi[0:1]
    mark = jnp.bitwise_and(flags, 1) > 0
    shl = jnp.bitwise_and(flags, 2) > 0
    shr = jnp.logical_and(jnp.logical_not(shl), jnp.bitwise_and(flags, 4) > 0)
    active = jnp.logical_and(mark, jnp.logical_or(shl, shr))

    def first_set(row):
        v = mi[row:row + 1]
        low = jnp.bitwise_and(v, -v)
        f = low.astype(jnp.float32)
        e = jnp.right_shift(jax.lax.bitcast_convert_type(f, jnp.int32), 23) - 127
        return jnp.where(v == 0, 0, e)

    lo = first_set(1)
    hi = first_set(2)
    sa = first_set(3)

    value = lo + 16 * hi
    shl_res = jnp.bitwise_and(jnp.left_shift(value, sa), 255)
    shr_res = jnp.right_shift(value, sa)
    res = jnp.where(shl, shl_res, shr_res)
    res_lo = jnp.bitwise_and(res, 15)
    res_hi = 16 + jnp.right_shift(res, 4)

    p = jnp.where(
        active,
        jnp.bitwise_or(jnp.left_shift(1, res_lo), jnp.left_shift(1, res_hi)),
        0)
    row = jax.lax.broadcasted_iota(jnp.int32, (32, t), 0)
    pb = jnp.bitwise_and(jnp.right_shift(p, row), 1).astype(jnp.bfloat16)
    add = jax.lax.dot_general(
        pb, i2_ref[...], (((0,), (0,)), ((), ())),
        preferred_element_type=jnp.float32)

    o_ref[:, 0:96] = x[:, 0:96]
    o_ref[:, 96:128] = x[:, 96:128] + add
    o_ref[:, 128:512] = x[:, 128:512]


def kernel(x_bd):
    b, s, d = x_bd.shape
    n = b * s
    x2 = x_bd.reshape(n, d)
    out = pl.pallas_call(
        _body,
        grid=(n // _TOK_BLK,),
        in_specs=[
            pl.BlockSpec((_TOK_BLK, d), lambda i: (i, 0)),
            pl.BlockSpec((8, 64), lambda i: (0, 0)),
            pl.BlockSpec((32, 32), lambda i: (0, 0)),
        ],
        out_specs=pl.BlockSpec((_TOK_BLK, d), lambda i: (i, 0)),
        out_shape=jax.ShapeDtypeStruct((n, d), x_bd.dtype),
    )(x2, _WT, _I2)
    return out.reshape(b, s, d)
